# baseline (speedup 1.0000x reference)
import functools

import jax
import jax.numpy as jnp
from jax import lax
from jax.experimental import pallas as pl
from jax.experimental.pallas import tpu as pltpu
from jax.experimental.pallas import tpu_sc as plsc

_E = 160000
_A = 640000
_LANES = 16
_NCORES = 2
_NSUB = 16
_HALF = _E // _NCORES
_PER_SUB = _A // _NSUB
_CHUNK = 2000
_NCHUNK = _PER_SUB // _CHUNK
_ZROWS = 5008
_SEG_ROWS = _NSUB * _ZROWS
_DISCARD = _HALF


_PBLK = 1000
_PGRID = (_A // 8) // _PBLK
_PROWS = _A // 2


def _proj_body(*refs):
    a_refs, w_ref, o_ref = refs[:4], refs[4], refs[5]
    parts = [
        lax.dot_general(a_ref[...], w_ref[...],
                        (((1,), (0,)), ((), ())),
                        preferred_element_type=jnp.float32)
        for a_ref in a_refs
    ]
    o_ref[...] = jnp.concatenate(parts, axis=1)


def _project(a2, w98):
    def _mk(j):
        return pl.BlockSpec((_PBLK, a2.shape[1]),
                            lambda i, j=j: (j * _PGRID + i, 0))

    return pl.pallas_call(
        _proj_body,
        grid=(_PGRID,),
        in_specs=[_mk(j) for j in range(4)] + [
            pl.BlockSpec(w98.shape, lambda i: (0, 0)),
        ],
        out_specs=pl.BlockSpec((_PBLK, 128), lambda i: (i, 0)),
        out_shape=jax.ShapeDtypeStruct((_A // 8, 128), jnp.float32),
    )(*([a2] * 4 + [w98]))


def _segsum(t16, kj_idx):
    mesh = plsc.VectorSubcoreMesh(core_axis_name="c", subcore_axis_name="s")

    @functools.partial(
        pl.kernel,
        out_type=jax.ShapeDtypeStruct((_E, _LANES), jnp.float32),
        mesh=mesh,
        scratch_types=[
            pltpu.VMEM((_CHUNK, _LANES), jnp.float32),
            pltpu.VMEM((_CHUNK,), jnp.int32),
            pltpu.VMEM_SHARED((_SEG_ROWS, _LANES), jnp.float32),
        ],
        compiler_params=pltpu.CompilerParams(use_tc_tiling_on_sc=False),
    )
    def k(t_hbm, i_hbm, seg_hbm, tbuf, ibuf, seg_sh):
        c = lax.axis_index("c")
        s = lax.axis_index("s")

        zvec = jnp.zeros((_LANES,), jnp.float32)

        @pl.loop(0, _CHUNK)
        def _(i):
            tbuf[i] = zvec

        zbase = s * _ZROWS
        pltpu.sync_copy(tbuf.at[pl.ds(0, _CHUNK)], seg_sh.at[pl.ds(zbase, _CHUNK)])
        pltpu.sync_copy(tbuf.at[pl.ds(0, _CHUNK)],
                        seg_sh.at[pl.ds(zbase + _CHUNK, _CHUNK)])
        pltpu.sync_copy(tbuf.at[pl.ds(0, _ZROWS - 2 * _CHUNK)],
                        seg_sh.at[pl.ds(zbase + 2 * _CHUNK, _ZROWS - 2 * _CHUNK)])
        plsc.subcore_barrier()

        lo = c * _HALF

        @pl.loop(0, _NCHUNK)
        def _(j):
            off = s * _PER_SUB + j * _CHUNK
            pltpu.sync_copy(t_hbm.at[pl.ds(off, _CHUNK)], tbuf)
            pltpu.sync_copy(i_hbm.at[pl.ds(off, _CHUNK)], ibuf)

            @pl.loop(0, _CHUNK, step=_LANES)
            def _(i):
                v = ibuf[pl.ds(i, _LANES)] - lo
                ok = (v >= 0) & (v < _HALF)
                ibuf[pl.ds(i, _LANES)] = jnp.where(ok, v, _DISCARD)

            pltpu.sync_copy(tbuf, seg_sh.at[ibuf], add=True)

        plsc.subcore_barrier()

        out_rows = _HALF // _NSUB
        ob = s * out_rows
        pltpu.sync_copy(seg_sh.at[pl.ds(ob, out_rows)],
                        seg_hbm.at[pl.ds(lo + ob, out_rows)])

    kj_p = jnp.transpose(jnp.reshape(kj_idx, (4, _A // 8, 2)),
                         (1, 0, 2)).reshape(_A)
    return k(jnp.reshape(t16, (_A, _LANES)), kj_p)


def _ef_body(m_ref, e_ref, wn_ref, b_ref, we_ref, o_ref):
    x = lax.dot_general(m_ref[...].astype(jnp.bfloat16), wn_ref[...],
                        (((1,), (1,)), ((), ())),
                        preferred_element_type=jnp.float32) + b_ref[...]
    nbr = x * jax.nn.sigmoid(x)
    te = lax.dot_general(e_ref[...].astype(jnp.bfloat16), we_ref[...],
                         (((1,), (1,)), ((), ())),
                         preferred_element_type=jnp.float32)
    o_ref[...] = (nbr * te).astype(jnp.bfloat16)


def _edge_factor(m_ji, e_rbf, w_nbr, b2, w_erbf):
    blk = 4000
    cat = m_ji.shape[1]
    nrbf = e_rbf.shape[1]
    return pl.pallas_call(
        _ef_body,
        grid=(_E // blk,),
        in_specs=[
            pl.BlockSpec((blk, cat), lambda i: (i, 0)),
            pl.BlockSpec((blk, nrbf), lambda i: (i, 0)),
            pl.BlockSpec((cat, cat), lambda i: (0, 0)),
            pl.BlockSpec((1, cat), lambda i: (0, 0)),
            pl.BlockSpec((cat, nrbf), lambda i: (0, 0)),
        ],
        out_specs=pl.BlockSpec((blk, cat), lambda i: (i, 0)),
        out_shape=jax.ShapeDtypeStruct((_E, cat), jnp.bfloat16),
    )(m_ji, e_rbf, w_nbr, b2, w_erbf)


def _final_body(ef_ref, s_ref, fw_ref, o_ref):
    sa = lax.dot_general(s_ref[...].astype(jnp.bfloat16), fw_ref[...],
                         (((1,), (0,)), ((), ())),
                         preferred_element_type=jnp.float32)
    o_ref[...] = ef_ref[...].astype(jnp.float32) * sa


def _final(ef, seg, fw16):
    blk = 4000
    cat = ef.shape[1]
    return pl.pallas_call(
        _final_body,
        grid=(_E // blk,),
        in_specs=[
            pl.BlockSpec((blk, cat), lambda i: (i, 0)),
            pl.BlockSpec((blk, _LANES), lambda i: (i, 0)),
            pl.BlockSpec((_LANES, cat), lambda i: (0, 0)),
        ],
        out_specs=pl.BlockSpec((blk, cat), lambda i: (i, 0)),
        out_shape=jax.ShapeDtypeStruct((_E, cat), jnp.float32),
    )(ef, seg, fw16)


def kernel(m_ji, nbr_list, angle_list, e_rbf, a_sbf, kj_idx,
           W_nbr, b_nbr, W_erbf, W_asbf, final_w):
    cat = m_ji.shape[1]
    nsph = W_asbf.shape[1]
    wt = jnp.zeros((nsph, _LANES), jnp.float32).at[:, :W_asbf.shape[0]].set(W_asbf.T)
    w98 = (jnp.zeros((2 * nsph, 2 * _LANES), jnp.float32)
           .at[:nsph, :_LANES].set(wt)
           .at[nsph:, _LANES:].set(wt).astype(jnp.bfloat16))
    fw16 = (jnp.zeros((_LANES, cat), jnp.float32)
            .at[:final_w.shape[0]].set(final_w).astype(jnp.bfloat16))
    a2 = jnp.reshape(a_sbf, (_PROWS, 2 * nsph)).astype(jnp.bfloat16)
    t16 = _project(a2, w98)
    seg = _segsum(t16, kj_idx)
    ef = _edge_factor(m_ji, e_rbf, W_nbr.astype(jnp.bfloat16),
                      b_nbr.reshape(1, cat), W_erbf.astype(jnp.bfloat16))
    return _final(ef, seg, fw16)


# speedup vs baseline: 1.4860x; 1.4860x over previous
import functools

import jax
import jax.numpy as jnp
from jax import lax
from jax.experimental import pallas as pl
from jax.experimental.pallas import tpu as pltpu
from jax.experimental.pallas import tpu_sc as plsc

_E = 160000
_A = 640000
_LANES = 16
_NCORES = 2
_NSUB = 16
_HALF = _E // _NCORES
_PER_SUB = _A // _NSUB
_CHUNK = 2000
_NCHUNK = _PER_SUB // _CHUNK
_ZROWS = 5008
_SEG_ROWS = _NSUB * _ZROWS
_DISCARD = _HALF


_PBLK = 320
_PGRID = (_A // 8) // _PBLK


def _proj_body(*refs):
    a_refs, w_ref, o_ref = refs[:8], refs[8], refs[9]
    parts = [
        lax.dot_general(a_ref[...], w_ref[...],
                        (((1,), (1,)), ((), ())),
                        preferred_element_type=jnp.float32)
        for a_ref in a_refs
    ]
    o_ref[...] = jnp.concatenate(parts, axis=1)


def _project(a_sbf, w16):
    nsph = a_sbf.shape[1]

    def _mk(j):
        return pl.BlockSpec((_PBLK, nsph), lambda i, j=j: (j * _PGRID + i, 0))

    return pl.pallas_call(
        _proj_body,
        grid=(_PGRID,),
        in_specs=[_mk(j) for j in range(8)] + [
            pl.BlockSpec(w16.shape, lambda i: (0, 0)),
        ],
        out_specs=pl.BlockSpec((_PBLK, 128), lambda i: (i, 0)),
        out_shape=jax.ShapeDtypeStruct((_A // 8, 128), jnp.float32),
    )(*([a_sbf] * 8 + [w16]))


def _segsum(t16, kj_idx):
    mesh = plsc.VectorSubcoreMesh(core_axis_name="c", subcore_axis_name="s")

    @functools.partial(
        pl.kernel,
        out_type=jax.ShapeDtypeStruct((_E, _LANES), jnp.float32),
        mesh=mesh,
        scratch_types=[
            pltpu.VMEM((_CHUNK, _LANES), jnp.float32),
            pltpu.VMEM((_CHUNK,), jnp.int32),
            pltpu.VMEM_SHARED((_SEG_ROWS, _LANES), jnp.float32),
        ],
        compiler_params=pltpu.CompilerParams(use_tc_tiling_on_sc=False),
    )
    def k(t_hbm, i_hbm, seg_hbm, tbuf, ibuf, seg_sh):
        c = lax.axis_index("c")
        s = lax.axis_index("s")

        zvec = jnp.zeros((_LANES,), jnp.float32)

        @pl.loop(0, _CHUNK)
        def _(i):
            tbuf[i] = zvec

        zbase = s * _ZROWS
        pltpu.sync_copy(tbuf.at[pl.ds(0, _CHUNK)], seg_sh.at[pl.ds(zbase, _CHUNK)])
        pltpu.sync_copy(tbuf.at[pl.ds(0, _CHUNK)],
                        seg_sh.at[pl.ds(zbase + _CHUNK, _CHUNK)])
        pltpu.sync_copy(tbuf.at[pl.ds(0, _ZROWS - 2 * _CHUNK)],
                        seg_sh.at[pl.ds(zbase + 2 * _CHUNK, _ZROWS - 2 * _CHUNK)])
        plsc.subcore_barrier()

        lo = c * _HALF

        @pl.loop(0, _NCHUNK)
        def _(j):
            off = s * _PER_SUB + j * _CHUNK
            pltpu.sync_copy(t_hbm.at[pl.ds(off, _CHUNK)], tbuf)
            pltpu.sync_copy(i_hbm.at[pl.ds(off, _CHUNK)], ibuf)

            @pl.loop(0, _CHUNK, step=_LANES)
            def _(i):
                v = ibuf[pl.ds(i, _LANES)] - lo
                ok = (v >= 0) & (v < _HALF)
                ibuf[pl.ds(i, _LANES)] = jnp.where(ok, v, _DISCARD)

            pltpu.sync_copy(tbuf, seg_sh.at[ibuf], add=True)

        plsc.subcore_barrier()

        out_rows = _HALF // _NSUB
        ob = s * out_rows
        pltpu.sync_copy(seg_sh.at[pl.ds(ob, out_rows)],
                        seg_hbm.at[pl.ds(lo + ob, out_rows)])

    kj_p = jnp.transpose(jnp.reshape(kj_idx, (8, _A // 8))).reshape(_A)
    return k(jnp.reshape(t16, (_A, _LANES)), kj_p)


def _ef_body(m_ref, e_ref, wn_ref, b_ref, we_ref, o_ref):
    x = lax.dot_general(m_ref[...].astype(jnp.bfloat16), wn_ref[...],
                        (((1,), (1,)), ((), ())),
                        preferred_element_type=jnp.float32) + b_ref[...]
    nbr = x * jax.nn.sigmoid(x)
    te = lax.dot_general(e_ref[...].astype(jnp.bfloat16), we_ref[...],
                         (((1,), (1,)), ((), ())),
                         preferred_element_type=jnp.float32)
    o_ref[...] = (nbr * te).astype(jnp.bfloat16)


def _edge_factor(m_ji, e_rbf, w_nbr, b2, w_erbf):
    blk = 4000
    cat = m_ji.shape[1]
    nrbf = e_rbf.shape[1]
    return pl.pallas_call(
        _ef_body,
        grid=(_E // blk,),
        in_specs=[
            pl.BlockSpec((blk, cat), lambda i: (i, 0)),
            pl.BlockSpec((blk, nrbf), lambda i: (i, 0)),
            pl.BlockSpec((cat, cat), lambda i: (0, 0)),
            pl.BlockSpec((1, cat), lambda i: (0, 0)),
            pl.BlockSpec((cat, nrbf), lambda i: (0, 0)),
        ],
        out_specs=pl.BlockSpec((blk, cat), lambda i: (i, 0)),
        out_shape=jax.ShapeDtypeStruct((_E, cat), jnp.bfloat16),
    )(m_ji, e_rbf, w_nbr, b2, w_erbf)


def _final_body(ef_ref, s_ref, fw_ref, o_ref):
    sa = lax.dot_general(s_ref[...].astype(jnp.bfloat16), fw_ref[...],
                         (((1,), (0,)), ((), ())),
                         preferred_element_type=jnp.float32)
    o_ref[...] = ef_ref[...].astype(jnp.float32) * sa


def _final(ef, seg, fw16):
    blk = 4000
    cat = ef.shape[1]
    return pl.pallas_call(
        _final_body,
        grid=(_E // blk,),
        in_specs=[
            pl.BlockSpec((blk, cat), lambda i: (i, 0)),
            pl.BlockSpec((blk, _LANES), lambda i: (i, 0)),
            pl.BlockSpec((_LANES, cat), lambda i: (0, 0)),
        ],
        out_specs=pl.BlockSpec((blk, cat), lambda i: (i, 0)),
        out_shape=jax.ShapeDtypeStruct((_E, cat), jnp.float32),
    )(ef, seg, fw16)


def kernel(m_ji, nbr_list, angle_list, e_rbf, a_sbf, kj_idx,
           W_nbr, b_nbr, W_erbf, W_asbf, final_w):
    cat = m_ji.shape[1]
    w16 = (jnp.zeros((_LANES, W_asbf.shape[1]), jnp.float32)
           .at[:W_asbf.shape[0]].set(W_asbf).astype(jnp.bfloat16))
    fw16 = (jnp.zeros((_LANES, cat), jnp.float32)
            .at[:final_w.shape[0]].set(final_w).astype(jnp.bfloat16))
    t16 = _project(a_sbf.astype(jnp.bfloat16), w16)
    seg = _segsum(t16, kj_idx)
    ef = _edge_factor(m_ji, e_rbf, W_nbr.astype(jnp.bfloat16),
                      b_nbr.reshape(1, cat), W_erbf.astype(jnp.bfloat16))
    return _final(ef, seg, fw16)
